# baseline (speedup 1.0000x reference)
import functools

import jax
import jax.numpy as jnp
from jax import lax
from jax.experimental import pallas as pl
from jax.experimental.pallas import tpu as pltpu
from jax.experimental.pallas import tpu_sc as plsc

N, E, D, H, DE, A, B = 10000, 320000, 128, 128, 16, 32, 8
NC, NS, L = 2, 16, 16
NW = NC * NS
EPW = E // NW
CH = 80
NCHUNK = EPW // CH
RPT = 640
RZB = 80


def _leaky(x):
    return jnp.where(x >= 0, x, 0.01 * x)


NSLOT = 4


@functools.cache
def _get_sc_aggregate():
    mesh = plsc.VectorSubcoreMesh(core_axis_name="c", subcore_axis_name="s",
                                  num_cores=NC, num_subcores=NS)
    return functools.partial(
        pl.kernel,
        out_type=jax.ShapeDtypeStruct((NC, N, D), jnp.float32),
        mesh=mesh,
        scratch_types=(
            [pltpu.VMEM((CH,), jnp.int32) for _ in range(NSLOT)]
            + [pltpu.VMEM((CH,), jnp.int32) for _ in range(NSLOT)]
            + [pltpu.VMEM((CH, D), jnp.float32) for _ in range(NSLOT)]
            + [pltpu.VMEM_SHARED((N, D), jnp.float32)]
            + [pltpu.SemaphoreType.DMA for _ in range(4 * NSLOT)]
        ),
    )(_sc_aggregate_body)


def _sc_aggregate_body(h_hbm, src_hbm, dst_hbm, e_hbm, out_hbm, *scr):
    sidx = scr[0:NSLOT]
    didx = scr[NSLOT:2 * NSLOT]
    ebuf = scr[2 * NSLOT:3 * NSLOT]
    accum = scr[3 * NSLOT]
    sems = scr[3 * NSLOT + 1:]
    zbuf = ebuf[0]
    semio = sems[0:NSLOT]
    seme = sems[NSLOT:2 * NSLOT]
    semg = sems[2 * NSLOT:3 * NSLOT]
    semsc = sems[3 * NSLOT:4 * NSLOT]

    cid = lax.axis_index("c")
    sid = lax.axis_index("s")
    wid = cid * NS + sid

    def _zrow(i, c):
        for j in range(D // L):
            zbuf[i, pl.ds(j * L, L)] = jnp.zeros((L,), jnp.float32)
        return c
    lax.fori_loop(0, RZB, _zrow, 0)
    row0 = sid * RPT
    nblk = jnp.where(sid == NS - 1, 5, RPT // RZB)

    def _zcopy(k, c):
        pltpu.sync_copy(zbuf, accum.at[pl.ds(row0 + k * RZB, RZB)])
        return c
    lax.fori_loop(0, nblk, _zcopy, 0)
    plsc.subcore_barrier()

    base_w = wid * EPW

    def _idx_descs(c, s):
        base = base_w + c * CH
        return (
            pltpu.make_async_copy(src_hbm.at[pl.ds(base, CH)], sidx[s], semio[s]),
            pltpu.make_async_copy(dst_hbm.at[pl.ds(base, CH)], didx[s], semio[s]),
        )

    def _e_desc(c, s):
        base = base_w + c * CH
        return pltpu.make_async_copy(e_hbm.at[pl.ds(base, CH)], ebuf[s], seme[s])

    def _gadd_desc(s):
        return pltpu.make_async_copy(h_hbm.at[sidx[s]], ebuf[s], semg[s])

    def _sc_desc(s):
        return pltpu.make_async_copy(ebuf[s], accum.at[didx[s]], semsc[s])

    for c in (0, 1):
        for d_ in _idx_descs(c, c):
            d_.start()
        _e_desc(c, c).start()
    for d_ in _idx_descs(0, 0):
        d_.wait()
    _e_desc(0, 0).wait()
    _gadd_desc(0).start(add=True)

    def _half(i, j):
        @pl.when(i <= NCHUNK - 1)
        def _():
            s1 = (j + 1) % NSLOT
            s2 = (j + 2) % NSLOT

            @pl.when(i >= 2)
            def _():
                _sc_desc(s2).wait()

            @pl.when(i + 2 <= NCHUNK - 1)
            def _():
                for d_ in _idx_descs(i + 2, s2):
                    d_.start()
                _e_desc(i + 2, s2).start()

            @pl.when(i + 1 <= NCHUNK - 1)
            def _():
                for d_ in _idx_descs(i + 1, s1):
                    d_.wait()
                _e_desc(i + 1, s1).wait()
                _gadd_desc(s1).start(add=True)

            _gadd_desc(j).wait()

            @plsc.parallel_loop(0, CH, step=1, unroll=4)
            def _row(r):
                for jj in range(D // L):
                    sl = pl.ds(jj * L, L)
                    ebuf[j][r, sl] = jnp.maximum(ebuf[j][r, sl], 0.0)

            _sc_desc(j).start(add=True)

    def _quad(k, c):
        for j in range(NSLOT):
            _half(NSLOT * k + j, j)
        return c
    lax.fori_loop(0, (NCHUNK + NSLOT - 1) // NSLOT, _quad, 0)
    _sc_desc((NCHUNK - 2) % NSLOT).wait()
    _sc_desc((NCHUNK - 1) % NSLOT).wait()
    plsc.subcore_barrier()

    def _wcopy(k, c):
        r = row0 + k * RZB
        pltpu.sync_copy(accum.at[pl.ds(r, RZB)], zbuf)
        pltpu.sync_copy(zbuf, out_hbm.at[cid, pl.ds(r, RZB)])
        return c
    lax.fori_loop(0, nblk, _wcopy, 0)


_EBLK = 6400


def _edge_emb(edge_attr, We, be):
    def body(a_ref, w_ref, b_ref, o_ref):
        o_ref[...] = jnp.dot(a_ref[...], w_ref[...],
                             preferred_element_type=jnp.float32) + b_ref[...]

    return pl.pallas_call(
        body,
        grid=(E // _EBLK,),
        in_specs=[
            pl.BlockSpec((_EBLK, DE), lambda i: (i, 0)),
            pl.BlockSpec((DE, D), lambda i: (0, 0)),
            pl.BlockSpec((1, D), lambda i: (0, 0)),
        ],
        out_specs=pl.BlockSpec((_EBLK, D), lambda i: (i, 0)),
        out_shape=jax.ShapeDtypeStruct((E, D), jnp.float32),
    )(edge_attr, We, be.reshape(1, D))


def _node_mlp(h, p0, p1, Wa, ba, Wb, bb, final_relu):
    def body(h_ref, p0_ref, p1_ref, wa_ref, ba_ref, wb_ref, bb_ref, o_ref):
        u = h_ref[...] + p0_ref[...] + p1_ref[...]
        u = _leaky(jnp.dot(u, wa_ref[...],
                           preferred_element_type=jnp.float32) + ba_ref[...])
        u = _leaky(jnp.dot(u, wb_ref[...],
                           preferred_element_type=jnp.float32) + bb_ref[...])
        if final_relu:
            u = jnp.maximum(u, 0.0)
        o_ref[...] = u

    return pl.pallas_call(
        body,
        out_shape=jax.ShapeDtypeStruct((N, D), jnp.float32),
    )(h, p0, p1, Wa, ba.reshape(1, D), Wb, bb.reshape(1, D))


def _tail(h1, q0, q1, W2a, b2a, W2b, b2b, Wa1, ba1, Wa2, ba2,
          Wn1, bn1, Wn2, bn2, Wn3, bn3):
    def body(h_ref, q0_ref, q1_ref, w2a_ref, b2a_ref, w2b_ref, b2b_ref,
             wa1_ref, ba1_ref, wa2_ref, ba2_ref,
             wn1_ref, bn1_ref, wn2_ref, bn2_ref, wn3_ref, bn3_ref,
             act_ref, s_ref):
        u = h_ref[...] + q0_ref[...] + q1_ref[...]
        u = _leaky(jnp.dot(u, w2a_ref[...],
                           preferred_element_type=jnp.float32) + b2a_ref[...])
        h2 = _leaky(jnp.dot(u, w2b_ref[...],
                            preferred_element_type=jnp.float32) + b2b_ref[...])

        grp = lax.broadcasted_iota(jnp.int32, (B, N), 0)
        row = lax.broadcasted_iota(jnp.int32, (B, N), 1) // (N // B)
        sel = jnp.where(grp == row, 1.0 / (N // B), 0.0)
        xm = jnp.dot(sel, h2, preferred_element_type=jnp.float32)

        a = _leaky(jnp.dot(xm, wa1_ref[...],
                           preferred_element_type=jnp.float32) + ba1_ref[...])
        lg = _leaky(jnp.dot(a, wa2_ref[...],
                            preferred_element_type=jnp.float32) + ba2_ref[...])
        z = lg - jnp.max(lg, axis=1, keepdims=True)
        ez = jnp.exp(z)
        act_ref[...] = ez / jnp.sum(ez, axis=1, keepdims=True)

        v = _leaky(jnp.dot(h2, wn1_ref[...],
                           preferred_element_type=jnp.float32) + bn1_ref[...])
        v = _leaky(jnp.dot(v, wn2_ref[...],
                           preferred_element_type=jnp.float32) + bn2_ref[...])
        r = jnp.sum(v * wn3_ref[...], axis=1, keepdims=True) + bn3_ref[...]
        s_ref[...] = 1.0 / (1.0 + jnp.exp(-r))

    return pl.pallas_call(
        body,
        out_shape=[jax.ShapeDtypeStruct((B, A), jnp.float32),
                   jax.ShapeDtypeStruct((N, 1), jnp.float32)],
    )(h1, q0, q1, W2a, b2a.reshape(1, D), W2b, b2b.reshape(1, D),
      Wa1, ba1.reshape(1, H), Wa2, ba2.reshape(1, A),
      Wn1, bn1.reshape(1, H), Wn2, bn2.reshape(1, H),
      Wn3.reshape(1, H), bn3.reshape(1, 1))


def kernel(x, edge_index, edge_attr, We1, be1, W1a, b1a, W1b, b1b,
           We2, be2, W2a, b2a, W2b, b2b, Wa1, ba1, Wa2, ba2,
           Wn1, bn1, Wn2, bn2, Wn3, bn3):
    src = edge_index[0]
    dst = edge_index[1]
    e1 = _edge_emb(edge_attr, We1, be1)
    e2 = _edge_emb(edge_attr, We2, be2)

    p = _get_sc_aggregate()(x, src, dst, e1)
    h1 = _node_mlp(x, p[0], p[1], W1a, b1a, W1b, b1b, final_relu=True)
    q = _get_sc_aggregate()(h1, src, dst, e2)

    action_prob, s = _tail(h1, q[0], q[1], W2a, b2a, W2b, b2b,
                           Wa1, ba1, Wa2, ba2, Wn1, bn1, Wn2, bn2, Wn3, bn3)
    node_scores = s[:, 0].reshape(N // B, B).T
    return (action_prob, node_scores)


# speedup vs baseline: 1.0331x; 1.0331x over previous
import functools

import jax
import jax.numpy as jnp
from jax import lax
from jax.experimental import pallas as pl
from jax.experimental.pallas import tpu as pltpu
from jax.experimental.pallas import tpu_sc as plsc

N, E, D, H, DE, A, B = 10000, 320000, 128, 128, 16, 32, 8
NC, NS, L = 2, 16, 16
NW = NC * NS
EPW = E // NW
CH = 80
NCHUNK = EPW // CH
RPT = 640
RZB = 80


def _leaky(x):
    return jnp.where(x >= 0, x, 0.01 * x)


NSLOT = 4


@functools.cache
def _get_sc_aggregate():
    mesh = plsc.VectorSubcoreMesh(core_axis_name="c", subcore_axis_name="s",
                                  num_cores=NC, num_subcores=NS)
    return functools.partial(
        pl.kernel,
        out_type=jax.ShapeDtypeStruct((NC, N, D), jnp.float32),
        mesh=mesh,
        scratch_types=(
            [pltpu.VMEM((CH,), jnp.int32) for _ in range(NSLOT)]
            + [pltpu.VMEM((CH,), jnp.int32) for _ in range(NSLOT)]
            + [pltpu.VMEM((CH, D), jnp.float32) for _ in range(NSLOT)]
            + [pltpu.VMEM_SHARED((N, D), jnp.float32)]
            + [pltpu.SemaphoreType.DMA for _ in range(4 * NSLOT)]
        ),
    )(_sc_aggregate_body)


def _sc_aggregate_body(h_hbm, src_hbm, dst_hbm, e_hbm, out_hbm, *scr):
    sidx = scr[0:NSLOT]
    didx = scr[NSLOT:2 * NSLOT]
    ebuf = scr[2 * NSLOT:3 * NSLOT]
    accum = scr[3 * NSLOT]
    sems = scr[3 * NSLOT + 1:]
    zbuf = ebuf[0]
    semio = sems[0:NSLOT]
    seme = sems[NSLOT:2 * NSLOT]
    semg = sems[2 * NSLOT:3 * NSLOT]
    semsc = sems[3 * NSLOT:4 * NSLOT]

    cid = lax.axis_index("c")
    sid = lax.axis_index("s")
    wid = cid * NS + sid

    def _zrow(i, c):
        for j in range(D // L):
            zbuf[i, pl.ds(j * L, L)] = jnp.zeros((L,), jnp.float32)
        return c
    lax.fori_loop(0, RZB, _zrow, 0)
    row0 = sid * RPT
    nblk = jnp.where(sid == NS - 1, 5, RPT // RZB)

    def _zcopy(k, c):
        pltpu.sync_copy(zbuf, accum.at[pl.ds(row0 + k * RZB, RZB)])
        return c
    lax.fori_loop(0, nblk, _zcopy, 0)
    plsc.subcore_barrier()

    base_w = wid * EPW

    def _idx_descs(c, s):
        base = base_w + c * CH
        return (
            pltpu.make_async_copy(src_hbm.at[pl.ds(base, CH)], sidx[s], semio[s]),
            pltpu.make_async_copy(dst_hbm.at[pl.ds(base, CH)], didx[s], semio[s]),
        )

    def _e_desc(c, s):
        base = base_w + c * CH
        return pltpu.make_async_copy(e_hbm.at[pl.ds(base, CH)], ebuf[s], seme[s])

    def _gadd_desc(s):
        return pltpu.make_async_copy(h_hbm.at[sidx[s]], ebuf[s], semg[s])

    def _sc_desc(s):
        return pltpu.make_async_copy(ebuf[s], accum.at[didx[s]], semsc[s])

    for c in (0, 1):
        for d_ in _idx_descs(c, c):
            d_.start()
        _e_desc(c, c).start()
    for d_ in _idx_descs(0, 0):
        d_.wait()
    _e_desc(0, 0).wait()
    _gadd_desc(0).start(add=True)

    def _half(i, j):
        @pl.when(i <= NCHUNK - 1)
        def _():
            s1 = (j + 1) % NSLOT
            s2 = (j + 2) % NSLOT

            @pl.when(i >= 2)
            def _():
                _sc_desc(s2).wait()

            @pl.when(i + 2 <= NCHUNK - 1)
            def _():
                for d_ in _idx_descs(i + 2, s2):
                    d_.start()
                _e_desc(i + 2, s2).start()

            @pl.when(i + 1 <= NCHUNK - 1)
            def _():
                for d_ in _idx_descs(i + 1, s1):
                    d_.wait()
                _e_desc(i + 1, s1).wait()
                _gadd_desc(s1).start(add=True)

            _gadd_desc(j).wait()

            @plsc.parallel_loop(0, CH, step=1, unroll=4)
            def _row(r):
                for jj in range(D // L):
                    sl = pl.ds(jj * L, L)
                    ebuf[j][r, sl] = jnp.maximum(ebuf[j][r, sl], 0.0)

            _sc_desc(j).start(add=True)

    def _quad(k, c):
        for j in range(NSLOT):
            _half(NSLOT * k + j, j)
        return c
    lax.fori_loop(0, (NCHUNK + NSLOT - 1) // NSLOT, _quad, 0)
    _sc_desc((NCHUNK - 2) % NSLOT).wait()
    _sc_desc((NCHUNK - 1) % NSLOT).wait()
    plsc.subcore_barrier()

    def _wcopy(k, c):
        r = row0 + k * RZB
        pltpu.sync_copy(accum.at[pl.ds(r, RZB)], zbuf)
        pltpu.sync_copy(zbuf, out_hbm.at[cid, pl.ds(r, RZB)])
        return c
    lax.fori_loop(0, nblk, _wcopy, 0)


_EBLK = 6400


def _edge_emb(edge_attr, We1, be1, We2, be2):
    def body(a_ref, w1_ref, b1_ref, w2_ref, b2_ref, o1_ref, o2_ref):
        a = a_ref[...].astype(jnp.bfloat16)
        o1_ref[...] = jnp.dot(a, w1_ref[...].astype(jnp.bfloat16),
                              preferred_element_type=jnp.float32) + b1_ref[...]
        o2_ref[...] = jnp.dot(a, w2_ref[...].astype(jnp.bfloat16),
                              preferred_element_type=jnp.float32) + b2_ref[...]

    return pl.pallas_call(
        body,
        grid=(E // _EBLK,),
        in_specs=[
            pl.BlockSpec((_EBLK, DE), lambda i: (i, 0)),
            pl.BlockSpec((DE, D), lambda i: (0, 0)),
            pl.BlockSpec((1, D), lambda i: (0, 0)),
            pl.BlockSpec((DE, D), lambda i: (0, 0)),
            pl.BlockSpec((1, D), lambda i: (0, 0)),
        ],
        out_specs=[pl.BlockSpec((_EBLK, D), lambda i: (i, 0))] * 2,
        out_shape=[jax.ShapeDtypeStruct((E, D), jnp.float32)] * 2,
    )(edge_attr, We1, be1.reshape(1, D), We2, be2.reshape(1, D))


def _node_mlp(h, p0, p1, Wa, ba, Wb, bb, final_relu):
    def body(h_ref, p0_ref, p1_ref, wa_ref, ba_ref, wb_ref, bb_ref, o_ref):
        u = h_ref[...] + p0_ref[...] + p1_ref[...]
        u = _leaky(jnp.dot(u, wa_ref[...],
                           preferred_element_type=jnp.float32) + ba_ref[...])
        u = _leaky(jnp.dot(u, wb_ref[...],
                           preferred_element_type=jnp.float32) + bb_ref[...])
        if final_relu:
            u = jnp.maximum(u, 0.0)
        o_ref[...] = u

    return pl.pallas_call(
        body,
        out_shape=jax.ShapeDtypeStruct((N, D), jnp.float32),
    )(h, p0, p1, Wa, ba.reshape(1, D), Wb, bb.reshape(1, D))


def _tail(h1, q0, q1, W2a, b2a, W2b, b2b, Wa1, ba1, Wa2, ba2,
          Wn1, bn1, Wn2, bn2, Wn3, bn3):
    def body(h_ref, q0_ref, q1_ref, w2a_ref, b2a_ref, w2b_ref, b2b_ref,
             wa1_ref, ba1_ref, wa2_ref, ba2_ref,
             wn1_ref, bn1_ref, wn2_ref, bn2_ref, wn3_ref, bn3_ref,
             act_ref, s_ref):
        u = h_ref[...] + q0_ref[...] + q1_ref[...]
        u = _leaky(jnp.dot(u, w2a_ref[...],
                           preferred_element_type=jnp.float32) + b2a_ref[...])
        h2 = _leaky(jnp.dot(u, w2b_ref[...],
                            preferred_element_type=jnp.float32) + b2b_ref[...])

        grp = lax.broadcasted_iota(jnp.int32, (B, N), 0)
        row = lax.broadcasted_iota(jnp.int32, (B, N), 1) // (N // B)
        sel = jnp.where(grp == row, 1.0 / (N // B), 0.0)
        xm = jnp.dot(sel, h2, preferred_element_type=jnp.float32)

        a = _leaky(jnp.dot(xm, wa1_ref[...],
                           preferred_element_type=jnp.float32) + ba1_ref[...])
        lg = _leaky(jnp.dot(a, wa2_ref[...],
                            preferred_element_type=jnp.float32) + ba2_ref[...])
        z = lg - jnp.max(lg, axis=1, keepdims=True)
        ez = jnp.exp(z)
        act_ref[...] = ez / jnp.sum(ez, axis=1, keepdims=True)

        v = _leaky(jnp.dot(h2, wn1_ref[...],
                           preferred_element_type=jnp.float32) + bn1_ref[...])
        v = _leaky(jnp.dot(v, wn2_ref[...],
                           preferred_element_type=jnp.float32) + bn2_ref[...])
        r = jnp.sum(v * wn3_ref[...], axis=1, keepdims=True) + bn3_ref[...]
        s_ref[...] = 1.0 / (1.0 + jnp.exp(-r))

    return pl.pallas_call(
        body,
        out_shape=[jax.ShapeDtypeStruct((B, A), jnp.float32),
                   jax.ShapeDtypeStruct((N, 1), jnp.float32)],
    )(h1, q0, q1, W2a, b2a.reshape(1, D), W2b, b2b.reshape(1, D),
      Wa1, ba1.reshape(1, H), Wa2, ba2.reshape(1, A),
      Wn1, bn1.reshape(1, H), Wn2, bn2.reshape(1, H),
      Wn3.reshape(1, H), bn3.reshape(1, 1))


def kernel(x, edge_index, edge_attr, We1, be1, W1a, b1a, W1b, b1b,
           We2, be2, W2a, b2a, W2b, b2b, Wa1, ba1, Wa2, ba2,
           Wn1, bn1, Wn2, bn2, Wn3, bn3):
    src = edge_index[0]
    dst = edge_index[1]
    e1, e2 = _edge_emb(edge_attr, We1, be1, We2, be2)

    p = _get_sc_aggregate()(x, src, dst, e1)
    h1 = _node_mlp(x, p[0], p[1], W1a, b1a, W1b, b1b, final_relu=True)
    q = _get_sc_aggregate()(h1, src, dst, e2)

    action_prob, s = _tail(h1, q[0], q[1], W2a, b2a, W2b, b2b,
                           Wa1, ba1, Wa2, ba2, Wn1, bn1, Wn2, bn2, Wn3, bn3)
    node_scores = s[:, 0].reshape(N // B, B).T
    return (action_prob, node_scores)
